# baseline (speedup 1.0000x reference)
import functools

import jax
import jax.numpy as jnp
from jax import lax
from jax.experimental import pallas as pl
from jax.experimental.pallas import tpu as pltpu
from jax.experimental.pallas import tpu_sc as plsc

N = 10000
D = 256
E = 160000
NC = 2
NS = 16
NW = NC * NS
LANES = 16
WIN = 320
AROWS = WIN + 1
BLK = 1600
VPB = BLK // LANES
NBLK = E // BLK
NB2 = NBLK // 2
CAP = 5504
PEND = CAP + 32
SLOTS = 32
ROUNDS = CAP // SLOTS
R2 = ROUNDS // 2
ROW_VECS = D // LANES


def _sc_body(x_hbm, meta_hbm, out_hbm,
             mb_a, mb_b, pend_col, pend_loc, pend_w, sbuf, cntbuf,
             gb_a, gb_b, acc, sxa, sxb, sma, smb):
    c = lax.axis_index("c")
    s = lax.axis_index("s")
    wid = s * NC + c
    base_row = wid * WIN

    zeros16 = jnp.zeros((LANES,), jnp.float32)
    zeros16i = jnp.zeros((LANES,), jnp.int32)

    def zero_acc_row(r, _):
        for j in range(ROW_VECS):
            acc[r, pl.ds(j * LANES, LANES)] = zeros16
        return 0

    lax.fori_loop(0, AROWS, zero_acc_row, 0)

    def init_pend(k, _):
        iota16 = lax.iota(jnp.int32, LANES)
        sl = pl.ds(k * LANES, LANES)
        slot = k * LANES + iota16
        pend_col[sl] = (wid * 300 + slot) % N
        pend_loc[sl] = zeros16i + WIN
        pend_w[sl] = zeros16
        return 0

    lax.fori_loop(0, PEND // LANES, init_pend, 0)
    cntbuf[pl.ds(0, LANES)] = zeros16i
    sbuf[pl.ds(0, LANES)] = zeros16i
    sbuf[pl.ds(2 * LANES, LANES)] = zeros16i

    def issue_m(b, mb, sm):
        pltpu.async_copy(meta_hbm.at[b], mb, sm)

    def wait_m(b, mb, sm):
        pltpu.make_async_copy(meta_hbm.at[b], mb, sm).wait()

    def scan_block(b, mb):
        off = b * BLK

        def vreg_body(k, _):
            iota16 = lax.iota(jnp.int32, LANES)
            sl = pl.ds(k * LANES, LANES)
            cnt_vec = cntbuf[pl.ds(0, LANES)]
            local = mb[0, sl] - base_row
            m = (local >= 0) & (local < WIN)
            mi = jnp.where(m, 1, 0)
            v = mi
            for d in (1, 2, 4, 8):
                sbuf[pl.ds(LANES, LANES)] = v
                v = v + sbuf[pl.ds(LANES - d, LANES)]
            csum = v
            u = mi
            for d in (1, 2, 4, 8):
                sbuf[pl.ds(LANES, LANES)] = u
                u = u + sbuf[pl.ds(LANES + d, LANES)]
            npend = csum + u - mi
            pos_own = jnp.minimum(csum + (cnt_vec - 1), CAP + iota16)
            pos = jnp.where(m, pos_own, (CAP + LANES) + iota16)
            plsc.store_scatter(pend_col, [pos], mb[1, sl])
            plsc.store_scatter(pend_loc, [pos], local)
            plsc.store_scatter(pend_w, [pos],
                               plsc.bitcast(mb[2, sl], jnp.float32))
            cntbuf[pl.ds(0, LANES)] = cnt_vec + npend
            return 0

        lax.fori_loop(0, VPB, vreg_body, 0)

    issue_m(0, mb_a, sma)

    def block2_body(b2, _):
        ba = 2 * b2
        issue_m(ba + 1, mb_b, smb)
        wait_m(ba, mb_a, sma)
        scan_block(ba, mb_a)
        lax.cond(ba + 2 < NBLK,
                 lambda: issue_m(ba + 2, mb_a, sma),
                 lambda: None)
        wait_m(ba + 1, mb_b, smb)
        scan_block(ba + 1, mb_b)
        return 0

    lax.fori_loop(0, NB2, block2_body, 0)

    def issue(rr, gb, sx):
        csl = pend_col.at[pl.ds(rr * SLOTS, SLOTS)]
        pltpu.async_copy(x_hbm.at[csl], gb, sx)

    def wait(rr, gb, sx):
        csl = pend_col.at[pl.ds(rr * SLOTS, SLOTS)]
        pltpu.make_async_copy(x_hbm.at[csl], gb, sx).wait()

    def process(rr, gb):
        base = rr * SLOTS

        def edge_body(e, _):
            iota16 = lax.iota(jnp.int32, LANES)
            lr = plsc.load_gather(pend_loc, [zeros16i + (base + e)])
            wv = plsc.load_gather(pend_w, [zeros16i + (base + e)])
            for j in range(ROW_VECS):
                plsc.addupdate_scatter(
                    acc, [lr, iota16 + j * LANES],
                    gb[e, pl.ds(j * LANES, LANES)] * wv)
            return 0

        lax.fori_loop(0, SLOTS, edge_body, 0)

    issue(0, gb_a, sxa)

    def round2_body(r2, _):
        ra = 2 * r2
        issue(ra + 1, gb_b, sxb)
        wait(ra, gb_a, sxa)
        process(ra, gb_a)
        lax.cond(ra + 2 < ROUNDS,
                 lambda: issue(ra + 2, gb_a, sxa),
                 lambda: None)
        wait(ra + 1, gb_b, sxb)
        process(ra + 1, gb_b)
        return 0

    lax.fori_loop(0, R2, round2_body, 0)

    tail = N - (NW - 1) * WIN

    def write_last():
        pltpu.sync_copy(acc.at[pl.ds(0, tail)],
                        out_hbm.at[pl.ds(base_row, tail)])

    def write_full():
        pltpu.sync_copy(acc.at[pl.ds(0, WIN)],
                        out_hbm.at[pl.ds(base_row, WIN)])

    lax.cond(wid == NW - 1, write_last, write_full)


def _sc_call(x, meta):
    return pl.kernel(
        _sc_body,
        out_type=jax.ShapeDtypeStruct((N, D), jnp.float32),
        mesh=plsc.VectorSubcoreMesh(core_axis_name="c", subcore_axis_name="s"),
        compiler_params=pltpu.CompilerParams(needs_layout_passes=False),
        scratch_types=[
            pltpu.VMEM((3, BLK), jnp.int32),
            pltpu.VMEM((3, BLK), jnp.int32),
            pltpu.VMEM((PEND,), jnp.int32),
            pltpu.VMEM((PEND,), jnp.int32),
            pltpu.VMEM((PEND,), jnp.float32),
            pltpu.VMEM((3 * LANES,), jnp.int32),
            pltpu.VMEM((LANES,), jnp.int32),
            pltpu.VMEM((SLOTS, D), jnp.float32),
            pltpu.VMEM((SLOTS, D), jnp.float32),
            pltpu.VMEM((AROWS, D), jnp.float32),
            pltpu.SemaphoreType.DMA,
            pltpu.SemaphoreType.DMA,
            pltpu.SemaphoreType.DMA,
            pltpu.SemaphoreType.DMA,
        ],
    )(x, meta)


def _mm_body(a_ref, w_ref, b_ref, o_ref):
    o_ref[...] = jnp.dot(a_ref[...], w_ref[...],
                         preferred_element_type=jnp.float32) + b_ref[...]


def _matmul_bias(agg, weight, bias2d):
    blk = 400
    return pl.pallas_call(
        _mm_body,
        grid=(N // blk,),
        in_specs=[
            pl.BlockSpec((blk, D), lambda i: (i, 0)),
            pl.BlockSpec((D, D), lambda i: (0, 0)),
            pl.BlockSpec((1, D), lambda i: (0, 0)),
        ],
        out_specs=pl.BlockSpec((blk, D), lambda i: (i, 0)),
        out_shape=jax.ShapeDtypeStruct((N, D), jnp.float32),
    )(agg, weight, bias2d)


def kernel(inputs, edge_index, edge_weight, weight, bias):
    row = edge_index[0].astype(jnp.int32)
    col = edge_index[1].astype(jnp.int32)
    w_bits = lax.bitcast_convert_type(edge_weight, jnp.int32)
    meta = (jnp.stack([row, col, w_bits])
            .reshape(3, NBLK, BLK).transpose(1, 0, 2))
    agg = _sc_call(inputs, meta)
    return _matmul_bias(agg, weight, bias.reshape(1, D))


# speedup vs baseline: 1.3984x; 1.3984x over previous
import functools

import jax
import jax.numpy as jnp
from jax import lax
from jax.experimental import pallas as pl
from jax.experimental.pallas import tpu as pltpu
from jax.experimental.pallas import tpu_sc as plsc

N = 10000
D = 256
E = 160000
NC = 2
NS = 16
NW = NC * NS
LANES = 16
WIN = 320
AROWS = WIN + 1
BLK = 1600
VPB = BLK // LANES
NBLK = E // BLK
NB2 = NBLK // 2
CAP = 5504
PEND = CAP + 32
SLOTS = 32
ROUNDS = CAP // SLOTS
R2 = ROUNDS // 2
ROW_VECS = D // LANES


def _sc_body(x_hbm, meta_hbm, out_hbm,
             mb_a, mb_b, pend_col, pend_loc, pend_w, tmp16, cntbuf,
             gb_a, gb_b, acc, sxa, sxb, sma, smb):
    c = lax.axis_index("c")
    s = lax.axis_index("s")
    wid = s * NC + c
    base_row = wid * WIN

    zeros16 = jnp.zeros((LANES,), jnp.float32)
    zeros16i = jnp.zeros((LANES,), jnp.int32)

    def zero_acc_row(r, _):
        for j in range(ROW_VECS):
            acc[r, pl.ds(j * LANES, LANES)] = zeros16
        return 0

    lax.fori_loop(0, AROWS, zero_acc_row, 0)

    def init_pend(k, _):
        iota16 = lax.iota(jnp.int32, LANES)
        sl = pl.ds(k * LANES, LANES)
        slot = k * LANES + iota16
        pend_col[sl] = (wid * 300 + slot) % N
        pend_loc[sl] = zeros16i + WIN
        pend_w[sl] = zeros16
        return 0

    lax.fori_loop(0, PEND // LANES, init_pend, 0)
    cntbuf[pl.ds(0, LANES)] = zeros16i

    def issue_m(b, mb, sm):
        pltpu.async_copy(meta_hbm.at[b], mb, sm)

    def wait_m(b, mb, sm):
        pltpu.make_async_copy(meta_hbm.at[b], mb, sm).wait()

    def scan_block(b, mb):
        off = b * BLK

        def vreg_body(k2, _):
            iota16 = lax.iota(jnp.int32, LANES)
            sl0 = pl.ds((2 * k2) * LANES, LANES)
            sl1 = pl.ds((2 * k2 + 1) * LANES, LANES)
            cnt_vec = cntbuf[pl.ds(0, LANES)]
            local0 = mb[0, sl0] - base_row
            local1 = mb[0, sl1] - base_row
            m0 = (local0 >= 0) & (local0 < WIN)
            m1 = (local1 >= 0) & (local1 < WIN)
            csum0 = plsc.cumsum(jnp.where(m0, 1, 0))
            csum1 = plsc.cumsum(jnp.where(m1, 1, 0))
            tmp16[pl.ds(0, LANES)] = csum0
            tmp16[pl.ds(LANES, LANES)] = csum1
            np0 = plsc.load_gather(tmp16, [zeros16i + (LANES - 1)])
            np1 = plsc.load_gather(tmp16, [zeros16i + (2 * LANES - 1)])
            pos0 = jnp.where(
                m0, jnp.minimum(csum0 + (cnt_vec - 1), CAP + iota16),
                (CAP + LANES) + iota16)
            cnt1 = cnt_vec + np0
            pos1 = jnp.where(
                m1, jnp.minimum(csum1 + (cnt1 - 1), CAP + iota16),
                (CAP + LANES) + iota16)
            plsc.store_scatter(pend_col, [pos0], mb[1, sl0])
            plsc.store_scatter(pend_loc, [pos0], local0)
            plsc.store_scatter(pend_w, [pos0],
                               plsc.bitcast(mb[2, sl0], jnp.float32))
            plsc.store_scatter(pend_col, [pos1], mb[1, sl1])
            plsc.store_scatter(pend_loc, [pos1], local1)
            plsc.store_scatter(pend_w, [pos1],
                               plsc.bitcast(mb[2, sl1], jnp.float32))
            cntbuf[pl.ds(0, LANES)] = cnt1 + np1
            return 0

        lax.fori_loop(0, VPB // 2, vreg_body, 0)

    issue_m(0, mb_a, sma)

    def block2_body(b2, _):
        ba = 2 * b2
        issue_m(ba + 1, mb_b, smb)
        wait_m(ba, mb_a, sma)
        scan_block(ba, mb_a)
        lax.cond(ba + 2 < NBLK,
                 lambda: issue_m(ba + 2, mb_a, sma),
                 lambda: None)
        wait_m(ba + 1, mb_b, smb)
        scan_block(ba + 1, mb_b)
        return 0

    lax.fori_loop(0, NB2, block2_body, 0)

    def issue(rr, gb, sx):
        csl = pend_col.at[pl.ds(rr * SLOTS, SLOTS)]
        pltpu.async_copy(x_hbm.at[csl], gb, sx)

    def wait(rr, gb, sx):
        csl = pend_col.at[pl.ds(rr * SLOTS, SLOTS)]
        pltpu.make_async_copy(x_hbm.at[csl], gb, sx).wait()

    def process(rr, gb):
        base = rr * SLOTS

        def edge_body(e, _):
            iota16 = lax.iota(jnp.int32, LANES)
            lr = plsc.load_gather(pend_loc, [zeros16i + (base + e)])
            wv = plsc.load_gather(pend_w, [zeros16i + (base + e)])
            for j in range(ROW_VECS):
                plsc.addupdate_scatter(
                    acc, [lr, iota16 + j * LANES],
                    gb[e, pl.ds(j * LANES, LANES)] * wv)
            return 0

        lax.fori_loop(0, SLOTS, edge_body, 0)

    issue(0, gb_a, sxa)

    def round2_body(r2, _):
        ra = 2 * r2
        issue(ra + 1, gb_b, sxb)
        wait(ra, gb_a, sxa)
        process(ra, gb_a)
        lax.cond(ra + 2 < ROUNDS,
                 lambda: issue(ra + 2, gb_a, sxa),
                 lambda: None)
        wait(ra + 1, gb_b, sxb)
        process(ra + 1, gb_b)
        return 0

    lax.fori_loop(0, R2, round2_body, 0)

    tail = N - (NW - 1) * WIN

    def write_last():
        pltpu.sync_copy(acc.at[pl.ds(0, tail)],
                        out_hbm.at[pl.ds(base_row, tail)])

    def write_full():
        pltpu.sync_copy(acc.at[pl.ds(0, WIN)],
                        out_hbm.at[pl.ds(base_row, WIN)])

    lax.cond(wid == NW - 1, write_last, write_full)


def _sc_call(x, meta):
    return pl.kernel(
        _sc_body,
        out_type=jax.ShapeDtypeStruct((N, D), jnp.float32),
        mesh=plsc.VectorSubcoreMesh(core_axis_name="c", subcore_axis_name="s"),
        compiler_params=pltpu.CompilerParams(needs_layout_passes=False),
        scratch_types=[
            pltpu.VMEM((3, BLK), jnp.int32),
            pltpu.VMEM((3, BLK), jnp.int32),
            pltpu.VMEM((PEND,), jnp.int32),
            pltpu.VMEM((PEND,), jnp.int32),
            pltpu.VMEM((PEND,), jnp.float32),
            pltpu.VMEM((2 * LANES,), jnp.int32),
            pltpu.VMEM((LANES,), jnp.int32),
            pltpu.VMEM((SLOTS, D), jnp.float32),
            pltpu.VMEM((SLOTS, D), jnp.float32),
            pltpu.VMEM((AROWS, D), jnp.float32),
            pltpu.SemaphoreType.DMA,
            pltpu.SemaphoreType.DMA,
            pltpu.SemaphoreType.DMA,
            pltpu.SemaphoreType.DMA,
        ],
    )(x, meta)


def _mm_body(a_ref, w_ref, b_ref, o_ref):
    o_ref[...] = jnp.dot(a_ref[...], w_ref[...],
                         preferred_element_type=jnp.float32) + b_ref[...]


def _matmul_bias(agg, weight, bias2d):
    blk = 400
    return pl.pallas_call(
        _mm_body,
        grid=(N // blk,),
        in_specs=[
            pl.BlockSpec((blk, D), lambda i: (i, 0)),
            pl.BlockSpec((D, D), lambda i: (0, 0)),
            pl.BlockSpec((1, D), lambda i: (0, 0)),
        ],
        out_specs=pl.BlockSpec((blk, D), lambda i: (i, 0)),
        out_shape=jax.ShapeDtypeStruct((N, D), jnp.float32),
    )(agg, weight, bias2d)


def kernel(inputs, edge_index, edge_weight, weight, bias):
    row = edge_index[0].astype(jnp.int32)
    col = edge_index[1].astype(jnp.int32)
    w_bits = lax.bitcast_convert_type(edge_weight, jnp.int32)
    meta = (jnp.stack([row, col, w_bits])
            .reshape(3, NBLK, BLK).transpose(1, 0, 2))
    agg = _sc_call(inputs, meta)
    return _matmul_bias(agg, weight, bias.reshape(1, D))


# speedup vs baseline: 1.4674x; 1.0493x over previous
import functools

import jax
import jax.numpy as jnp
from jax import lax
from jax.experimental import pallas as pl
from jax.experimental.pallas import tpu as pltpu
from jax.experimental.pallas import tpu_sc as plsc

N = 10000
D = 256
E = 160000
NC = 2
NS = 16
NW = NC * NS
LANES = 16
WIN = 320
AROWS = WIN + 1
BLK = 1600
VPB = BLK // LANES
NBLK = E // BLK
NB2 = NBLK // 2
CAP = 5504
PEND = CAP + 32
SLOTS = 32
ROUNDS = CAP // SLOTS
R2 = ROUNDS // 2
ROW_VECS = D // LANES


def _sc_body(x_hbm, meta_hbm, out_hbm,
             mb_a, mb_b, pend_col, pend_loc, pend_w, tmp16, cntbuf,
             gb_a, gb_b, acc, sxa, sxb, sma, smb):
    c = lax.axis_index("c")
    s = lax.axis_index("s")
    wid = s * NC + c
    base_row = wid * WIN

    zeros16 = jnp.zeros((LANES,), jnp.float32)
    zeros16i = jnp.zeros((LANES,), jnp.int32)

    def zero_acc_row(r, _):
        for j in range(ROW_VECS):
            acc[r, pl.ds(j * LANES, LANES)] = zeros16
        return 0

    lax.fori_loop(0, AROWS, zero_acc_row, 0)

    def init_pend(k, _):
        iota16 = lax.iota(jnp.int32, LANES)
        sl = pl.ds(k * LANES, LANES)
        slot = k * LANES + iota16
        pend_col[sl] = (wid * 300 + slot) % N
        pend_loc[sl] = zeros16i + WIN
        pend_w[sl] = zeros16
        return 0

    lax.fori_loop(0, PEND // LANES, init_pend, 0)
    cntbuf[pl.ds(0, LANES)] = zeros16i

    def issue_m(b, mb, sm):
        pltpu.async_copy(meta_hbm.at[b], mb, sm)

    def wait_m(b, mb, sm):
        pltpu.make_async_copy(meta_hbm.at[b], mb, sm).wait()

    def scan_block(b, mb):
        off = b * BLK

        def vreg_body(k4, _):
            iota16 = lax.iota(jnp.int32, LANES)
            sls = [pl.ds((4 * k4 + i) * LANES, LANES) for i in range(4)]
            cnt_vec = cntbuf[pl.ds(0, LANES)]
            locals_ = [mb[0, s] - base_row for s in sls]
            ms = [(lo >= 0) & (lo < WIN) for lo in locals_]
            csums = [plsc.cumsum(jnp.where(mm, 1, 0)) for mm in ms]
            for i in range(4):
                tmp16[pl.ds(i * LANES, LANES)] = csums[i]
            nps = [plsc.load_gather(tmp16,
                                    [zeros16i + ((i + 1) * LANES - 1)])
                   for i in range(4)]
            cnt = cnt_vec
            for i in range(4):
                pos = jnp.where(
                    ms[i], jnp.minimum(csums[i] + (cnt - 1), CAP + iota16),
                    (CAP + LANES) + iota16)
                plsc.store_scatter(pend_col, [pos], mb[1, sls[i]])
                plsc.store_scatter(pend_loc, [pos], locals_[i])
                plsc.store_scatter(pend_w, [pos],
                                   plsc.bitcast(mb[2, sls[i]], jnp.float32))
                cnt = cnt + nps[i]
            cntbuf[pl.ds(0, LANES)] = cnt
            return 0

        lax.fori_loop(0, VPB // 4, vreg_body, 0)

    issue_m(0, mb_a, sma)

    def block2_body(b2, _):
        ba = 2 * b2
        issue_m(ba + 1, mb_b, smb)
        wait_m(ba, mb_a, sma)
        scan_block(ba, mb_a)
        lax.cond(ba + 2 < NBLK,
                 lambda: issue_m(ba + 2, mb_a, sma),
                 lambda: None)
        wait_m(ba + 1, mb_b, smb)
        scan_block(ba + 1, mb_b)
        return 0

    lax.fori_loop(0, NB2, block2_body, 0)

    def issue(rr, gb, sx):
        csl = pend_col.at[pl.ds(rr * SLOTS, SLOTS)]
        pltpu.async_copy(x_hbm.at[csl], gb, sx)

    def wait(rr, gb, sx):
        csl = pend_col.at[pl.ds(rr * SLOTS, SLOTS)]
        pltpu.make_async_copy(x_hbm.at[csl], gb, sx).wait()

    def process(rr, gb):
        base = rr * SLOTS

        def edge_body(e, _):
            iota16 = lax.iota(jnp.int32, LANES)
            lr = plsc.load_gather(pend_loc, [zeros16i + (base + e)])
            wv = plsc.load_gather(pend_w, [zeros16i + (base + e)])
            for j in range(ROW_VECS):
                plsc.addupdate_scatter(
                    acc, [lr, iota16 + j * LANES],
                    gb[e, pl.ds(j * LANES, LANES)] * wv)
            return 0

        lax.fori_loop(0, SLOTS, edge_body, 0)

    issue(0, gb_a, sxa)

    def round2_body(r2, _):
        ra = 2 * r2
        issue(ra + 1, gb_b, sxb)
        wait(ra, gb_a, sxa)
        process(ra, gb_a)
        lax.cond(ra + 2 < ROUNDS,
                 lambda: issue(ra + 2, gb_a, sxa),
                 lambda: None)
        wait(ra + 1, gb_b, sxb)
        process(ra + 1, gb_b)
        return 0

    lax.fori_loop(0, R2, round2_body, 0)

    tail = N - (NW - 1) * WIN

    def write_last():
        pltpu.sync_copy(acc.at[pl.ds(0, tail)],
                        out_hbm.at[pl.ds(base_row, tail)])

    def write_full():
        pltpu.sync_copy(acc.at[pl.ds(0, WIN)],
                        out_hbm.at[pl.ds(base_row, WIN)])

    lax.cond(wid == NW - 1, write_last, write_full)


def _sc_call(x, meta):
    return pl.kernel(
        _sc_body,
        out_type=jax.ShapeDtypeStruct((N, D), jnp.float32),
        mesh=plsc.VectorSubcoreMesh(core_axis_name="c", subcore_axis_name="s"),
        compiler_params=pltpu.CompilerParams(needs_layout_passes=False),
        scratch_types=[
            pltpu.VMEM((3, BLK), jnp.int32),
            pltpu.VMEM((3, BLK), jnp.int32),
            pltpu.VMEM((PEND,), jnp.int32),
            pltpu.VMEM((PEND,), jnp.int32),
            pltpu.VMEM((PEND,), jnp.float32),
            pltpu.VMEM((4 * LANES,), jnp.int32),
            pltpu.VMEM((LANES,), jnp.int32),
            pltpu.VMEM((SLOTS, D), jnp.float32),
            pltpu.VMEM((SLOTS, D), jnp.float32),
            pltpu.VMEM((AROWS, D), jnp.float32),
            pltpu.SemaphoreType.DMA,
            pltpu.SemaphoreType.DMA,
            pltpu.SemaphoreType.DMA,
            pltpu.SemaphoreType.DMA,
        ],
    )(x, meta)


def _mm_body(a_ref, w_ref, b_ref, o_ref):
    o_ref[...] = jnp.dot(a_ref[...], w_ref[...],
                         preferred_element_type=jnp.float32) + b_ref[...]


def _matmul_bias(agg, weight, bias2d):
    blk = 400
    return pl.pallas_call(
        _mm_body,
        grid=(N // blk,),
        in_specs=[
            pl.BlockSpec((blk, D), lambda i: (i, 0)),
            pl.BlockSpec((D, D), lambda i: (0, 0)),
            pl.BlockSpec((1, D), lambda i: (0, 0)),
        ],
        out_specs=pl.BlockSpec((blk, D), lambda i: (i, 0)),
        out_shape=jax.ShapeDtypeStruct((N, D), jnp.float32),
    )(agg, weight, bias2d)


def kernel(inputs, edge_index, edge_weight, weight, bias):
    row = edge_index[0].astype(jnp.int32)
    col = edge_index[1].astype(jnp.int32)
    w_bits = lax.bitcast_convert_type(edge_weight, jnp.int32)
    meta = (jnp.stack([row, col, w_bits])
            .reshape(3, NBLK, BLK).transpose(1, 0, 2))
    agg = _sc_call(inputs, meta)
    return _matmul_bias(agg, weight, bias.reshape(1, D))
